# Initial kernel scaffold; baseline (speedup 1.0000x reference)
import jax
import jax.numpy as jnp
from jax.experimental import pallas as pl


def kernel(x, edge_index, edge_weight, W_z, b_z, Wl_z, bl_z, W_r, b_r, Wl_r, bl_r, W_h, b_h, Wl_h, bl_h, att, W_out, b_out):
    raise NotImplementedError("write your pallas kernel here")


# speedup vs baseline: 124.4089x; 124.4089x over previous
import functools

import jax
import jax.numpy as jnp
from jax import lax
from jax.experimental import pallas as pl
from jax.experimental.pallas import tpu as pltpu
from jax.experimental.pallas import tpu_sc as plsc

NC = 2
NS = 16
NW = NC * NS
LB = 128
CH = 16
WH = 16


def _sc_mesh():
    return plsc.VectorSubcoreMesh(core_axis_name="c", subcore_axis_name="s")


def _make_deg_kernel(n_pad, b_t):
    rows_per_tile = n_pad // NS

    @functools.partial(
        pl.kernel,
        mesh=_sc_mesh(),
        compiler_params=pltpu.CompilerParams(use_tc_tiling_on_sc=False),
        out_type=jax.ShapeDtypeStruct((NC * n_pad,), jnp.float32),
        scratch_types=[
            pltpu.VMEM((b_t, LB), jnp.int32),
            pltpu.VMEM((LB,), jnp.float32),
            pltpu.VMEM((rows_per_tile,), jnp.float32),
            pltpu.VMEM_SHARED((n_pad,), jnp.float32),
        ],
    )
    def deg_kernel(dst_hbm, deg_hbm, dst_v, ones_v, zero_v, deg_sh):
        c = lax.axis_index("c")
        s = lax.axis_index("s")
        wid = c * NS + s

        def fill_ones(i, _):
            ones_v[pl.ds(i * 16, 16)] = jnp.ones((16,), jnp.float32)
            return 0
        lax.fori_loop(0, LB // 16, fill_ones, 0)

        def fill_zero(i, _):
            zero_v[pl.ds(i * 16, 16)] = jnp.zeros((16,), jnp.float32)
            return 0
        lax.fori_loop(0, rows_per_tile // 16, fill_zero, 0)

        pltpu.sync_copy(zero_v, deg_sh.at[pl.ds(s * rows_per_tile, rows_per_tile)])
        plsc.subcore_barrier()

        pltpu.sync_copy(dst_hbm.at[wid], dst_v)

        def body(b, _):
            pltpu.sync_copy(ones_v, deg_sh.at[dst_v.at[b]], add=True)
            return 0
        lax.fori_loop(0, b_t, body, 0)

        plsc.subcore_barrier()
        pltpu.sync_copy(
            deg_sh.at[pl.ds(s * rows_per_tile, rows_per_tile)],
            deg_hbm.at[pl.ds(c * n_pad + s * rows_per_tile, rows_per_tile)])

    return deg_kernel


def _make_edge_kernel(n_rows, n_pad, b_t):
    rows_per_tile = n_pad // NS

    @functools.partial(
        pl.kernel,
        mesh=_sc_mesh(),
        compiler_params=pltpu.CompilerParams(use_tc_tiling_on_sc=False),
        out_type=jax.ShapeDtypeStruct((NC, n_pad, WH), jnp.float32),
        scratch_types=[
            pltpu.VMEM((CH, LB), jnp.int32),
            pltpu.VMEM((CH, LB), jnp.int32),
            pltpu.VMEM((LB, WH), jnp.float32),
            pltpu.VMEM((LB, WH), jnp.float32),
            pltpu.VMEM_SHARED((n_pad, WH), jnp.float32),
            pltpu.SemaphoreType.DMA,
        ],
    )
    def edge_kernel(s_hbm, src_hbm, dst_hbm, acc_hbm,
                    src_v, dst_v, rows_v, zero_v, acc_sh, sem):
        c = lax.axis_index("c")
        s = lax.axis_index("s")
        s_half = s_hbm.at[pl.ds(c * n_rows, n_rows)]

        def fill_zero(i, _):
            zero_v[i, pl.ds(0, 16)] = jnp.zeros((16,), jnp.float32)
            return 0
        lax.fori_loop(0, LB, fill_zero, 0)

        def zero_acc(i, _):
            pltpu.sync_copy(zero_v, acc_sh.at[pl.ds(s * rows_per_tile + i * LB, LB)])
            return 0
        lax.fori_loop(0, rows_per_tile // LB, zero_acc, 0)
        plsc.subcore_barrier()

        def chunk_body(ci, _):
            pltpu.sync_copy(src_hbm.at[s, pl.ds(ci * CH, CH)], src_v)
            pltpu.sync_copy(dst_hbm.at[s, pl.ds(ci * CH, CH)], dst_v)

            def body(b, _):
                pltpu.async_copy(s_half.at[src_v.at[b]], rows_v, sem).wait()
                pltpu.sync_copy(rows_v, acc_sh.at[dst_v.at[b]], add=True)
                return 0
            lax.fori_loop(0, CH, body, 0)
            return 0
        lax.fori_loop(0, b_t // CH, chunk_body, 0)

        plsc.subcore_barrier()
        pltpu.sync_copy(
            acc_sh.at[pl.ds(s * rows_per_tile, rows_per_tile)],
            acc_hbm.at[c, pl.ds(s * rows_per_tile, rows_per_tile)])

    return edge_kernel


def _prep_body(degT_ref, x24_ref, s_ref, dis_ref, *, nb):
    half = pl.program_id(0) // nb
    deg = degT_ref[:, 0:1] + degT_ref[:, 1:2] + 1.0
    dis = 1.0 / jnp.sqrt(deg)
    dis_ref[:] = dis
    cb = x24_ref.shape[0]
    lo = x24_ref[:, 0:WH]
    hi = jnp.concatenate(
        [x24_ref[:, WH:24], jnp.zeros((cb, 2 * WH - 24), jnp.float32)], axis=1)
    s_ref[:] = jnp.where(half == 0, lo, hi) * dis


def _gate_body(a0_ref, a1_ref, s0_ref, s1_ref, dis_ref, wmat_ref, wout_ref,
               bout_ref, out_ref):
    dis = dis_ref[:]
    agg = jnp.concatenate(
        [(a0_ref[:] + s0_ref[:]) * dis, (a1_ref[:] + s1_ref[:]) * dis], axis=1)
    cb = agg.shape[0]
    hacc = jnp.zeros((cb, 32), jnp.float32)
    for t in range(12):
        c0 = agg[:, t:t + 1]
        c1 = agg[:, 12 + t:13 + t]
        zpre = c0 * wmat_ref[0:1, :] + c1 * wmat_ref[1:2, :] + wmat_ref[2:3, :]
        hpre = c0 * wmat_ref[3:4, :] + c1 * wmat_ref[4:5, :] + wmat_ref[5:6, :]
        z = jax.nn.sigmoid(zpre)
        h = jnp.tanh(hpre)
        hacc = hacc + wmat_ref[6:7, t:t + 1] * (1.0 - z) * h
    out_ref[:] = jnp.dot(jnp.maximum(hacc, 0.0), wout_ref[:],
                         preferred_element_type=jnp.float32) + bout_ref[0:1, :]


def kernel(x, edge_index, edge_weight, W_z, b_z, Wl_z, bl_z, W_r, b_r, Wl_r,
           bl_r, W_h, b_h, Wl_h, bl_h, att, W_out, b_out):
    n, f_in, periods = x.shape
    e = edge_index.shape[1]
    hid = W_z.shape[1]

    n_pad = ((n + 16 + NS * LB - 1) // (NS * LB)) * (NS * LB)
    b_t = ((e + NW * LB - 1) // (NW * LB) + 7) // 8 * 8
    e_pad = NW * b_t * LB
    b_t2 = e_pad // (NS * LB)

    src = edge_index[0]
    dst = edge_index[1]
    fill = jnp.arange(e_pad - e, dtype=jnp.int32) % 16
    src_p = jnp.concatenate([src, fill])
    dst_p = jnp.concatenate([dst, n + fill])

    deg1d = _make_deg_kernel(n_pad, b_t)(dst_p.reshape(NW, b_t, LB))
    degT = deg1d.reshape(NC, n_pad).T

    cb = 2000
    nb = n // cb
    x24 = x.reshape(n, f_in * periods)
    S, dis = pl.pallas_call(
        functools.partial(_prep_body, nb=nb),
        grid=(2 * nb,),
        in_specs=[
            pl.BlockSpec((cb, 2), lambda i: (i % nb, 0)),
            pl.BlockSpec((cb, f_in * periods), lambda i: (i % nb, 0)),
        ],
        out_specs=[
            pl.BlockSpec((cb, WH), lambda i: (i, 0)),
            pl.BlockSpec((cb, 1), lambda i: (i % nb, 0)),
        ],
        out_shape=[
            jax.ShapeDtypeStruct((2 * n, WH), jnp.float32),
            jax.ShapeDtypeStruct((n, 1), jnp.float32),
        ],
    )(degT, x24)

    accn = _make_edge_kernel(n, n_pad, b_t2)(
        S, src_p.reshape(NS, b_t2, LB), dst_p.reshape(NS, b_t2, LB))

    wlz = Wl_z[:hid]
    wlh = Wl_h[:hid]
    wz_eff = W_z @ wlz
    bz_eff = b_z @ wlz + bl_z
    wh_eff = W_h @ wlh
    bh_eff = b_h @ wlh + bl_h
    probs = jax.nn.softmax(att)
    wmat = jnp.stack([
        wz_eff[0], wz_eff[1], bz_eff,
        wh_eff[0], wh_eff[1], bh_eff,
        jnp.pad(probs, (0, hid - periods)),
        jnp.zeros((hid,), jnp.float32),
    ])

    out = pl.pallas_call(
        _gate_body,
        grid=(nb,),
        in_specs=[
            pl.BlockSpec((cb, WH), lambda i: (i, 0)),
            pl.BlockSpec((cb, WH), lambda i: (i, 0)),
            pl.BlockSpec((cb, WH), lambda i: (i, 0)),
            pl.BlockSpec((cb, WH), lambda i: (i + nb, 0)),
            pl.BlockSpec((cb, 1), lambda i: (i, 0)),
            pl.BlockSpec((8, hid), lambda i: (0, 0)),
            pl.BlockSpec((hid, periods), lambda i: (0, 0)),
            pl.BlockSpec((1, periods), lambda i: (0, 0)),
        ],
        out_specs=pl.BlockSpec((cb, periods), lambda i: (i, 0)),
        out_shape=jax.ShapeDtypeStruct((n, periods), jnp.float32),
    )(accn[0], accn[1], S, S, dis, wmat, W_out, b_out.reshape(1, periods))

    return out


# speedup vs baseline: 158.9346x; 1.2775x over previous
import functools

import jax
import jax.numpy as jnp
from jax import lax
from jax.experimental import pallas as pl
from jax.experimental.pallas import tpu as pltpu
from jax.experimental.pallas import tpu_sc as plsc

NC = 2
NS = 16
NW = NC * NS
LB = 128
CH = 16
NBUF = 4
WH = 16


def _sc_mesh():
    return plsc.VectorSubcoreMesh(core_axis_name="c", subcore_axis_name="s")


def _make_deg_kernel(n_pad, b_t):
    rows_per_tile = n_pad // NS

    @functools.partial(
        pl.kernel,
        mesh=_sc_mesh(),
        compiler_params=pltpu.CompilerParams(use_tc_tiling_on_sc=False),
        out_type=jax.ShapeDtypeStruct((NC * n_pad,), jnp.float32),
        scratch_types=[
            pltpu.VMEM((b_t, LB), jnp.int32),
            pltpu.VMEM((LB,), jnp.float32),
            pltpu.VMEM((rows_per_tile,), jnp.float32),
            pltpu.VMEM_SHARED((n_pad,), jnp.float32),
        ],
    )
    def deg_kernel(dst_hbm, deg_hbm, dst_v, ones_v, zero_v, deg_sh):
        c = lax.axis_index("c")
        s = lax.axis_index("s")
        wid = c * NS + s

        def fill_ones(i, _):
            ones_v[pl.ds(i * 16, 16)] = jnp.ones((16,), jnp.float32)
            return 0
        lax.fori_loop(0, LB // 16, fill_ones, 0)

        def fill_zero(i, _):
            zero_v[pl.ds(i * 16, 16)] = jnp.zeros((16,), jnp.float32)
            return 0
        lax.fori_loop(0, rows_per_tile // 16, fill_zero, 0)

        pltpu.sync_copy(zero_v, deg_sh.at[pl.ds(s * rows_per_tile, rows_per_tile)])
        plsc.subcore_barrier()

        pltpu.sync_copy(dst_hbm.at[wid], dst_v)

        def body(b, _):
            pltpu.sync_copy(ones_v, deg_sh.at[dst_v.at[b]], add=True)
            return 0
        lax.fori_loop(0, b_t, body, 0)

        plsc.subcore_barrier()
        pltpu.sync_copy(
            deg_sh.at[pl.ds(s * rows_per_tile, rows_per_tile)],
            deg_hbm.at[pl.ds(c * n_pad + s * rows_per_tile, rows_per_tile)])

    return deg_kernel


def _make_edge_kernel(n_rows, n_pad, b_t):
    rows_per_tile = n_pad // NS

    @functools.partial(
        pl.kernel,
        mesh=_sc_mesh(),
        compiler_params=pltpu.CompilerParams(use_tc_tiling_on_sc=False),
        out_type=jax.ShapeDtypeStruct((NC, n_pad, WH), jnp.float32),
        scratch_types=[
            pltpu.VMEM((CH, LB), jnp.int32),
            pltpu.VMEM((CH, LB), jnp.int32),
            pltpu.VMEM((NBUF, LB, WH), jnp.float32),
            pltpu.VMEM((LB, WH), jnp.float32),
            pltpu.VMEM_SHARED((n_pad, WH), jnp.float32),
            [pltpu.SemaphoreType.DMA] * NBUF,
            [pltpu.SemaphoreType.DMA] * NBUF,
        ],
    )
    def edge_kernel(s_hbm, src_hbm, dst_hbm, acc_hbm,
                    src_v, dst_v, rows_v, zero_v, acc_sh, gsems, ssems):
        c = lax.axis_index("c")
        s = lax.axis_index("s")
        s_half = s_hbm.at[pl.ds(c * n_rows, n_rows)]

        def fill_zero(i, _):
            zero_v[i, pl.ds(0, 16)] = jnp.zeros((16,), jnp.float32)
            return 0
        lax.fori_loop(0, LB, fill_zero, 0)

        def zero_acc(i, _):
            pltpu.sync_copy(zero_v, acc_sh.at[pl.ds(s * rows_per_tile + i * LB, LB)])
            return 0
        lax.fori_loop(0, rows_per_tile // LB, zero_acc, 0)
        plsc.subcore_barrier()

        def wait_scat(k):
            pltpu.make_async_copy(
                rows_v.at[k], acc_sh.at[dst_v.at[0]], ssems[k]).wait()

        def wait_gath(k, b):
            pltpu.make_async_copy(
                s_half.at[src_v.at[b]], rows_v.at[k], gsems[k]).wait()

        def chunk_body(ci, _):
            @pl.when(ci > 0)
            def _():
                for k in range(NBUF):
                    wait_scat(k)
            pltpu.sync_copy(src_hbm.at[s, pl.ds(ci * CH, CH)], src_v)
            pltpu.sync_copy(dst_hbm.at[s, pl.ds(ci * CH, CH)], dst_v)
            for k in range(NBUF):
                pltpu.async_copy(
                    s_half.at[src_v.at[k]], rows_v.at[k], gsems[k])

            def group_body(j, _):
                for k in range(NBUF):
                    b = j * NBUF + k
                    wait_gath(k, b)
                    pltpu.async_copy(
                        rows_v.at[k], acc_sh.at[dst_v.at[b]], ssems[k],
                        add=True)

                @pl.when(j < CH // NBUF - 1)
                def _():
                    for k in range(NBUF):
                        wait_scat(k)
                        pltpu.async_copy(
                            s_half.at[src_v.at[(j + 1) * NBUF + k]],
                            rows_v.at[k], gsems[k])
                return 0
            lax.fori_loop(0, CH // NBUF, group_body, 0)
            return 0
        lax.fori_loop(0, b_t // CH, chunk_body, 0)
        for k in range(NBUF):
            wait_scat(k)

        plsc.subcore_barrier()
        pltpu.sync_copy(
            acc_sh.at[pl.ds(s * rows_per_tile, rows_per_tile)],
            acc_hbm.at[c, pl.ds(s * rows_per_tile, rows_per_tile)])

    return edge_kernel


def _prep_body(degT_ref, x24_ref, s_ref, dis_ref, *, nb):
    half = pl.program_id(0) // nb
    deg = degT_ref[:, 0:1] + degT_ref[:, 1:2] + 1.0
    dis = 1.0 / jnp.sqrt(deg)
    dis_ref[:] = dis
    cb = x24_ref.shape[0]
    lo = x24_ref[:, 0:WH]
    hi = jnp.concatenate(
        [x24_ref[:, WH:24], jnp.zeros((cb, 2 * WH - 24), jnp.float32)], axis=1)
    s_ref[:] = jnp.where(half == 0, lo, hi) * dis


def _gate_body(a0_ref, a1_ref, s0_ref, s1_ref, dis_ref, wmat_ref, wout_ref,
               bout_ref, out_ref):
    dis = dis_ref[:]
    agg = jnp.concatenate(
        [(a0_ref[:] + s0_ref[:]) * dis, (a1_ref[:] + s1_ref[:]) * dis], axis=1)
    cb = agg.shape[0]
    hacc = jnp.zeros((cb, 32), jnp.float32)
    for t in range(12):
        c0 = agg[:, t:t + 1]
        c1 = agg[:, 12 + t:13 + t]
        zpre = c0 * wmat_ref[0:1, :] + c1 * wmat_ref[1:2, :] + wmat_ref[2:3, :]
        hpre = c0 * wmat_ref[3:4, :] + c1 * wmat_ref[4:5, :] + wmat_ref[5:6, :]
        z = jax.nn.sigmoid(zpre)
        h = jnp.tanh(hpre)
        hacc = hacc + wmat_ref[6:7, t:t + 1] * (1.0 - z) * h
    out_ref[:] = jnp.dot(jnp.maximum(hacc, 0.0), wout_ref[:],
                         preferred_element_type=jnp.float32) + bout_ref[0:1, :]


def kernel(x, edge_index, edge_weight, W_z, b_z, Wl_z, bl_z, W_r, b_r, Wl_r,
           bl_r, W_h, b_h, Wl_h, bl_h, att, W_out, b_out):
    n, f_in, periods = x.shape
    e = edge_index.shape[1]
    hid = W_z.shape[1]

    n_pad = ((n + 16 + NS * LB - 1) // (NS * LB)) * (NS * LB)
    b_t = ((e + NW * LB - 1) // (NW * LB) + 7) // 8 * 8
    e_pad = NW * b_t * LB
    b_t2 = e_pad // (NS * LB)

    src = edge_index[0]
    dst = edge_index[1]
    fill = jnp.arange(e_pad - e, dtype=jnp.int32) % 16
    src_p = jnp.concatenate([src, fill])
    dst_p = jnp.concatenate([dst, n + fill])

    deg1d = _make_deg_kernel(n_pad, b_t)(dst_p.reshape(NW, b_t, LB))
    degT = deg1d.reshape(NC, n_pad).T

    cb = 2000
    nb = n // cb
    x24 = x.reshape(n, f_in * periods)
    S, dis = pl.pallas_call(
        functools.partial(_prep_body, nb=nb),
        grid=(2 * nb,),
        in_specs=[
            pl.BlockSpec((cb, 2), lambda i: (i % nb, 0)),
            pl.BlockSpec((cb, f_in * periods), lambda i: (i % nb, 0)),
        ],
        out_specs=[
            pl.BlockSpec((cb, WH), lambda i: (i, 0)),
            pl.BlockSpec((cb, 1), lambda i: (i % nb, 0)),
        ],
        out_shape=[
            jax.ShapeDtypeStruct((2 * n, WH), jnp.float32),
            jax.ShapeDtypeStruct((n, 1), jnp.float32),
        ],
    )(degT, x24)

    accn = _make_edge_kernel(n, n_pad, b_t2)(
        S, src_p.reshape(NS, b_t2, LB), dst_p.reshape(NS, b_t2, LB))

    wlz = Wl_z[:hid]
    wlh = Wl_h[:hid]
    wz_eff = W_z @ wlz
    bz_eff = b_z @ wlz + bl_z
    wh_eff = W_h @ wlh
    bh_eff = b_h @ wlh + bl_h
    probs = jax.nn.softmax(att)
    wmat = jnp.stack([
        wz_eff[0], wz_eff[1], bz_eff,
        wh_eff[0], wh_eff[1], bh_eff,
        jnp.pad(probs, (0, hid - periods)),
        jnp.zeros((hid,), jnp.float32),
    ])

    out = pl.pallas_call(
        _gate_body,
        grid=(nb,),
        in_specs=[
            pl.BlockSpec((cb, WH), lambda i: (i, 0)),
            pl.BlockSpec((cb, WH), lambda i: (i, 0)),
            pl.BlockSpec((cb, WH), lambda i: (i, 0)),
            pl.BlockSpec((cb, WH), lambda i: (i + nb, 0)),
            pl.BlockSpec((cb, 1), lambda i: (i, 0)),
            pl.BlockSpec((8, hid), lambda i: (0, 0)),
            pl.BlockSpec((hid, periods), lambda i: (0, 0)),
            pl.BlockSpec((1, periods), lambda i: (0, 0)),
        ],
        out_specs=pl.BlockSpec((cb, periods), lambda i: (i, 0)),
        out_shape=jax.ShapeDtypeStruct((n, periods), jnp.float32),
    )(accn[0], accn[1], S, S, dis, wmat, W_out, b_out.reshape(1, periods))

    return out
